# Initial kernel scaffold; baseline (speedup 1.0000x reference)
import jax
import jax.numpy as jnp
from jax import lax
from jax.experimental import pallas as pl
from jax.experimental.pallas import tpu as pltpu
from jax.experimental.pallas import tpu_sc as plsc

D = 768
DFF = 2048
E = 8
T = 2048
BLK = 128
NBLK = 40
TRASH = NBLK
NSLOT = (NBLK + 1) * BLK
MAXB = T // BLK

NC = 2
NS = 16
NW = NC * NS
TPW = T // NW
ROW16 = D // 16


def _erf(z):
    a = jnp.abs(z)
    t = 1.0 / (1.0 + 0.3275911 * a)
    poly = t * (0.254829592 + t * (-0.284496736 + t * (1.421413741
               + t * (-1.453152027 + t * 1.061405429))))
    e = 1.0 - poly * jnp.exp(-a * a)
    return jnp.where(z < 0, -e, e)


def _gelu(x):
    return 0.5 * x * (1.0 + _erf(x * 0.7071067811865476))


def _sigmoid(x):
    return 1.0 / (1.0 + jnp.exp(-x))


def _dot_t(a, b):
    return lax.dot_general(a, b, (((1,), (1,)), ((), ())),
                           preferred_element_type=jnp.float32)


def _router_body(x_ref, w1_ref, b1_ref, g_ref, be_ref, w2_ref, b2_ref,
                 ts0_ref, ts1_ref, v0_ref, v1_ref, binfo_ref):
    xf = x_ref[:]
    h = _dot_t(xf, w1_ref[:]) + b1_ref[:]
    mu = jnp.mean(h, axis=-1, keepdims=True)
    var = jnp.mean((h - mu) ** 2, axis=-1, keepdims=True)
    h = (h - mu) / jnp.sqrt(var + 1e-5) * g_ref[:] + be_ref[:]
    h = _gelu(h)
    logits = _dot_t(h, w2_ref[:]) + b2_ref[:]

    m = jnp.max(logits, axis=-1, keepdims=True)
    ex = jnp.exp(logits - m)
    p = ex / jnp.sum(ex, axis=-1, keepdims=True)

    lane = lax.broadcasted_iota(jnp.int32, (T, E), 1)
    v1 = jnp.max(p, axis=-1, keepdims=True)
    i1 = jnp.min(jnp.where(p == v1, lane, 127), axis=-1, keepdims=True)
    oh1 = lane == i1
    p2 = jnp.where(oh1, -1.0, p)
    v2 = jnp.max(p2, axis=-1, keepdims=True)
    i2 = jnp.min(jnp.where(p2 == v2, lane, 127), axis=-1, keepdims=True)
    oh2 = lane == i2

    sel = jnp.where(oh1 | oh2, 1.0, 0.0)

    r_i = lax.broadcasted_iota(jnp.int32, (T, T), 0)
    c_i = lax.broadcasted_iota(jnp.int32, (T, T), 1)
    tri = jnp.where(c_i < r_i, 1.0, 0.0).astype(jnp.bfloat16)
    rank = lax.dot_general(tri, sel.astype(jnp.bfloat16),
                           (((1,), (0,)), ((), ())),
                           preferred_element_type=jnp.float32)

    counts = jnp.sum(sel, axis=0, keepdims=True)
    pc = jnp.ceil(counts / BLK) * BLK
    ii = lax.broadcasted_iota(jnp.int32, (E, E), 0)
    jj = lax.broadcasted_iota(jnp.int32, (E, E), 1)
    cum = jnp.where(ii <= jj, 1.0, 0.0)
    ends = lax.dot_general(pc, cum, (((1,), (0,)), ((), ())),
                           preferred_element_type=jnp.float32)
    starts = ends - pc

    slot = starts + rank
    ts0 = jnp.sum(jnp.where(oh1, slot, 0.0), axis=-1, keepdims=True)
    ts1 = jnp.sum(jnp.where(oh2, slot, 0.0), axis=-1, keepdims=True)
    ts0_ref[:] = ts0.astype(jnp.int32)
    ts1_ref[:] = ts1.astype(jnp.int32)

    ones16 = jnp.ones((T, 16), jnp.float32)
    v0_ref[:] = v1 * ones16
    v1_ref[:] = v2 * ones16

    eye = jnp.where(ii == jj, 1.0, 0.0)
    starts_s = lax.dot_general(eye, starts, (((1,), (1,)), ((), ())),
                               preferred_element_type=jnp.float32)
    nb_s = lax.dot_general(eye, pc, (((1,), (1,)), ((), ())),
                           preferred_element_type=jnp.float32)
    binfo = jnp.concatenate([starts_s / BLK, nb_s / BLK], axis=1)
    binfo_ref[:] = binfo.astype(jnp.int32)


def _dispatch_body(x_hbm, ts0_hbm, ts1_hbm, xs_hbm, idx0, idx1, xbuf, sem):
    wid = lax.axis_index("s") * NC + lax.axis_index("c")
    base = wid * TPW
    pltpu.sync_copy(x_hbm.at[pl.ds(base, TPW)], xbuf)
    pltpu.sync_copy(ts0_hbm.at[pl.ds(base, TPW)], idx0)
    pltpu.sync_copy(ts1_hbm.at[pl.ds(base, TPW)], idx1)
    pltpu.async_copy(xbuf, xs_hbm.at[idx0], sem).wait()
    pltpu.async_copy(xbuf, xs_hbm.at[idx1], sem).wait()


def _run_dispatch(xf, ts0, ts1):
    mesh = plsc.VectorSubcoreMesh(core_axis_name="c", subcore_axis_name="s")
    f = pl.kernel(
        _dispatch_body,
        out_type=jax.ShapeDtypeStruct((NSLOT, D), jnp.float32),
        mesh=mesh,
        scratch_types=[
            pltpu.VMEM((TPW,), jnp.int32),
            pltpu.VMEM((TPW,), jnp.int32),
            pltpu.VMEM((TPW, D), jnp.float32),
            pltpu.SemaphoreType.DMA,
        ],
    )
    return f(xf, ts0, ts1)


def _slot_map(j, info):
    return (jnp.where(j < info[1], info[0] + j, TRASH), 0)


def _w_map(j, info):
    return (0, 0)


def _ffn0_body(info_ref, xs_ref, ys_in, w1, b1, w2, b2, ys_ref):
    @pl.when(pl.program_id(0) < info_ref[1])
    def _():
        h1 = _gelu(_dot_t(xs_ref[:], w1[:]) + b1[:])
        ys_ref[:] = _dot_t(h1, w2[:]) + b2[:]


def _ffn1_body(info_ref, xs_ref, ys_in, w1, b1, w2, b2, w3, b3, ys_ref):
    @pl.when(pl.program_id(0) < info_ref[1])
    def _():
        h1 = _dot_t(xs_ref[:], w1[:]) + b1[:]
        h1 = h1 * _sigmoid(h1)
        h2 = _dot_t(h1, w2[:]) + b2[:]
        ys_ref[:] = _dot_t(h2, w3[:]) + b3[:]


def _ffn2_body(info_ref, xs_ref, ys_in, w1, b1, w2, b2, w3, b3, ys_ref):
    @pl.when(pl.program_id(0) < info_ref[1])
    def _():
        h1 = jnp.maximum(_dot_t(xs_ref[:], w1[:]) + b1[:], 0.0)
        h2 = _sigmoid(_dot_t(h1, w2[:]) + b2[:])
        ys_ref[:] = _dot_t(h2, w3[:]) + b3[:]


_FFN_BODIES = {0: _ffn0_body, 1: _ffn1_body, 2: _ffn2_body}


def _run_ffn(ep_type, xs, ys, info, weights):
    w_specs = []
    w_args = []
    for (W, b) in weights:
        w_specs.append(pl.BlockSpec(W.shape, _w_map))
        w_specs.append(pl.BlockSpec(b.shape, _w_map))
        w_args.extend([W, b])
    grid_spec = pltpu.PrefetchScalarGridSpec(
        num_scalar_prefetch=1,
        grid=(MAXB,),
        in_specs=[
            pl.BlockSpec((BLK, D), _slot_map),
            pl.BlockSpec(memory_space=pltpu.MemorySpace.ANY),
        ] + w_specs,
        out_specs=pl.BlockSpec((BLK, D), _slot_map),
    )
    return pl.pallas_call(
        _FFN_BODIES[ep_type],
        grid_spec=grid_spec,
        out_shape=jax.ShapeDtypeStruct((NSLOT, D), jnp.float32),
        input_output_aliases={2: 0},
    )(info, xs, ys, *w_args)


def _combine_body(ys_hbm, ts0_hbm, ts1_hbm, v0_hbm, v1_hbm, out_hbm,
                  idx0, idx1, buf0, buf1, vb0, vb1, sem):
    wid = lax.axis_index("s") * NC + lax.axis_index("c")
    base = wid * TPW
    pltpu.sync_copy(ts0_hbm.at[pl.ds(base, TPW)], idx0)
    pltpu.sync_copy(ts1_hbm.at[pl.ds(base, TPW)], idx1)
    pltpu.async_copy(ys_hbm.at[idx0], buf0, sem).wait()
    pltpu.async_copy(ys_hbm.at[idx1], buf1, sem).wait()
    pltpu.sync_copy(v0_hbm.at[pl.ds(base, TPW)], vb0)
    pltpu.sync_copy(v1_hbm.at[pl.ds(base, TPW)], vb1)

    def row(t, carry):
        v0 = vb0[t]
        v1 = vb1[t]
        for c in range(ROW16):
            s = c * 16
            buf0[t, s:s + 16] = (v0 * buf0[t, s:s + 16]
                                 + v1 * buf1[t, s:s + 16])
        return carry

    lax.fori_loop(0, TPW, row, 0)
    pltpu.sync_copy(buf0, out_hbm.at[pl.ds(base, TPW)])


def _run_combine(ys, ts0, ts1, v0, v1):
    mesh = plsc.VectorSubcoreMesh(core_axis_name="c", subcore_axis_name="s")
    f = pl.kernel(
        _combine_body,
        out_type=jax.ShapeDtypeStruct((T, D), jnp.float32),
        mesh=mesh,
        scratch_types=[
            pltpu.VMEM((TPW,), jnp.int32),
            pltpu.VMEM((TPW,), jnp.int32),
            pltpu.VMEM((TPW, D), jnp.float32),
            pltpu.VMEM((TPW, D), jnp.float32),
            pltpu.VMEM((TPW, 16), jnp.float32),
            pltpu.VMEM((TPW, 16), jnp.float32),
            pltpu.SemaphoreType.DMA,
        ],
    )
    return f(ys, ts0, ts1, v0, v1)


def kernel(x, params):
    Bb, Ll, Dm = x.shape
    xf = x.reshape(T, D)
    r = params['router']
    temp = params['temp'][0]
    gate = params['gate']
    w2p = r['w2'] * (gate / temp)[:, None]
    b2p = (r['b2'] / temp + params['bias']) * gate

    ts0_2d, ts1_2d, v0, v1, binfo = pl.pallas_call(
        _router_body,
        out_shape=(
            jax.ShapeDtypeStruct((T, 1), jnp.int32),
            jax.ShapeDtypeStruct((T, 1), jnp.int32),
            jax.ShapeDtypeStruct((T, 16), jnp.float32),
            jax.ShapeDtypeStruct((T, 16), jnp.float32),
            jax.ShapeDtypeStruct((E, 2), jnp.int32),
        ),
    )(xf, r['w1'], r['b1'].reshape(1, -1), r['gamma'].reshape(1, -1),
      r['beta'].reshape(1, -1), w2p, b2p.reshape(1, -1))

    ts0 = ts0_2d.reshape(T)
    ts1 = ts1_2d.reshape(T)

    xs = _run_dispatch(xf, ts0, ts1)

    ys = jnp.zeros((NSLOT, D), jnp.float32)
    for e, ep in enumerate(params['experts']):
        t = e % 3
        if t == 0:
            weights = [(ep['l1'][0], ep['l1'][1].reshape(1, -1)),
                       (ep['l2'][0], ep['l2'][1].reshape(1, -1))]
        else:
            weights = [(ep['l1'][0], ep['l1'][1].reshape(1, -1)),
                       (ep['l2'][0], ep['l2'][1].reshape(1, -1)),
                       (ep['l3'][0], ep['l3'][1].reshape(1, -1))]
        ys = _run_ffn(t, xs, ys, binfo[e], weights)

    out = _run_combine(ys, ts0, ts1, v0, v1)
    return out.reshape(Bb, Ll, Dm)


# speedup vs baseline: 1.6408x; 1.6408x over previous
import jax
import jax.numpy as jnp
from jax import lax
from jax.experimental import pallas as pl
from jax.experimental.pallas import tpu as pltpu
from jax.experimental.pallas import tpu_sc as plsc

D = 768
DFF = 2048
E = 8
T = 2048
BLK = 128
NBLK = 40
TRASH = NBLK
NSLOT = (NBLK + 1) * BLK
MAXB = T // BLK

NC = 2
NS = 16
NW = NC * NS
TPW = T // NW
ROW16 = D // 16


def _erf(z):
    a = jnp.abs(z)
    t = 1.0 / (1.0 + 0.3275911 * a)
    poly = t * (0.254829592 + t * (-0.284496736 + t * (1.421413741
               + t * (-1.453152027 + t * 1.061405429))))
    e = 1.0 - poly * jnp.exp(-a * a)
    return jnp.where(z < 0, -e, e)


def _gelu(x):
    return 0.5 * x * (1.0 + _erf(x * 0.7071067811865476))


def _sigmoid(x):
    return 1.0 / (1.0 + jnp.exp(-x))


def _dot_t(a, b):
    return lax.dot_general(a, b, (((1,), (1,)), ((), ())),
                           preferred_element_type=jnp.float32)


def _router_body(x_ref, w1_ref, b1_ref, g_ref, be_ref, w2_ref, b2_ref,
                 ts0_ref, ts1_ref, v0_ref, v1_ref, binfo_ref):
    xf = x_ref[:]
    h = _dot_t(xf, w1_ref[:]) + b1_ref[:]
    mu = jnp.mean(h, axis=-1, keepdims=True)
    var = jnp.mean((h - mu) ** 2, axis=-1, keepdims=True)
    h = (h - mu) / jnp.sqrt(var + 1e-5) * g_ref[:] + be_ref[:]
    h = _gelu(h)
    logits = _dot_t(h, w2_ref[:]) + b2_ref[:]

    m = jnp.max(logits, axis=-1, keepdims=True)
    ex = jnp.exp(logits - m)
    p = ex / jnp.sum(ex, axis=-1, keepdims=True)

    lane = lax.broadcasted_iota(jnp.int32, (T, E), 1)
    v1 = jnp.max(p, axis=-1, keepdims=True)
    i1 = jnp.min(jnp.where(p == v1, lane, 127), axis=-1, keepdims=True)
    oh1 = lane == i1
    p2 = jnp.where(oh1, -1.0, p)
    v2 = jnp.max(p2, axis=-1, keepdims=True)
    i2 = jnp.min(jnp.where(p2 == v2, lane, 127), axis=-1, keepdims=True)
    oh2 = lane == i2

    sel = jnp.where(oh1 | oh2, 1.0, 0.0)

    r_i = lax.broadcasted_iota(jnp.int32, (T, T), 0)
    c_i = lax.broadcasted_iota(jnp.int32, (T, T), 1)
    tri = jnp.where(c_i < r_i, 1.0, 0.0).astype(jnp.bfloat16)
    rank = lax.dot_general(tri, sel.astype(jnp.bfloat16),
                           (((1,), (0,)), ((), ())),
                           preferred_element_type=jnp.float32)

    counts = jnp.sum(sel, axis=0, keepdims=True)
    pc = jnp.ceil(counts / BLK) * BLK
    ii = lax.broadcasted_iota(jnp.int32, (E, E), 0)
    jj = lax.broadcasted_iota(jnp.int32, (E, E), 1)
    cum = jnp.where(ii <= jj, 1.0, 0.0)
    ends = lax.dot_general(pc, cum, (((1,), (0,)), ((), ())),
                           preferred_element_type=jnp.float32)
    starts = ends - pc

    slot = starts + rank
    ts0 = jnp.sum(jnp.where(oh1, slot, 0.0), axis=-1, keepdims=True)
    ts1 = jnp.sum(jnp.where(oh2, slot, 0.0), axis=-1, keepdims=True)
    ts0_ref[:] = ts0.astype(jnp.int32)
    ts1_ref[:] = ts1.astype(jnp.int32)

    ones16 = jnp.ones((T, 16), jnp.float32)
    v0_ref[:] = v1 * ones16
    v1_ref[:] = v2 * ones16

    eye = jnp.where(ii == jj, 1.0, 0.0)
    starts_s = lax.dot_general(eye, starts, (((1,), (1,)), ((), ())),
                               preferred_element_type=jnp.float32)
    nb_s = lax.dot_general(eye, pc, (((1,), (1,)), ((), ())),
                           preferred_element_type=jnp.float32)
    binfo = jnp.concatenate([starts_s / BLK, nb_s / BLK], axis=1)
    binfo_ref[:] = binfo.astype(jnp.int32)


def _dispatch_body(x_hbm, ts0_hbm, ts1_hbm, xs_hbm, idx0, idx1, xbuf, sem):
    wid = lax.axis_index("s") * NC + lax.axis_index("c")
    base = wid * TPW
    pltpu.sync_copy(x_hbm.at[pl.ds(base, TPW)], xbuf)
    pltpu.sync_copy(ts0_hbm.at[pl.ds(base, TPW)], idx0)
    pltpu.sync_copy(ts1_hbm.at[pl.ds(base, TPW)], idx1)
    pltpu.async_copy(xbuf, xs_hbm.at[idx0], sem).wait()
    pltpu.async_copy(xbuf, xs_hbm.at[idx1], sem).wait()


def _run_dispatch(xf, ts0, ts1):
    mesh = plsc.VectorSubcoreMesh(core_axis_name="c", subcore_axis_name="s")
    f = pl.kernel(
        _dispatch_body,
        out_type=jax.ShapeDtypeStruct((NSLOT, D), jnp.float32),
        mesh=mesh,
        scratch_types=[
            pltpu.VMEM((TPW,), jnp.int32),
            pltpu.VMEM((TPW,), jnp.int32),
            pltpu.VMEM((TPW, D), jnp.float32),
            pltpu.SemaphoreType.DMA,
        ],
    )
    return f(xf, ts0, ts1)


def _slot_map(j, info):
    return (jnp.where(j < info[1], info[0] + j, TRASH), 0)


def _w_map(j, info):
    return (0, 0)


def _ffn0_body(info_ref, xs_ref, ys_in, w1, b1, w2, b2, ys_ref):
    @pl.when(pl.program_id(0) < info_ref[1])
    def _():
        h1 = _gelu(_dot_t(xs_ref[:], w1[:]) + b1[:])
        ys_ref[:] = _dot_t(h1, w2[:]) + b2[:]


def _ffn1_body(info_ref, xs_ref, ys_in, w1, b1, w2, b2, w3, b3, ys_ref):
    @pl.when(pl.program_id(0) < info_ref[1])
    def _():
        h1 = _dot_t(xs_ref[:], w1[:]) + b1[:]
        h1 = h1 * _sigmoid(h1)
        h2 = _dot_t(h1, w2[:]) + b2[:]
        ys_ref[:] = _dot_t(h2, w3[:]) + b3[:]


def _ffn2_body(info_ref, xs_ref, ys_in, w1, b1, w2, b2, w3, b3, ys_ref):
    @pl.when(pl.program_id(0) < info_ref[1])
    def _():
        h1 = jnp.maximum(_dot_t(xs_ref[:], w1[:]) + b1[:], 0.0)
        h2 = _sigmoid(_dot_t(h1, w2[:]) + b2[:])
        ys_ref[:] = _dot_t(h2, w3[:]) + b3[:]


_FFN_BODIES = {0: _ffn0_body, 1: _ffn1_body, 2: _ffn2_body}


def _run_ffn(ep_type, xs, ys, info, weights):
    w_specs = []
    w_args = []
    for (W, b) in weights:
        w_specs.append(pl.BlockSpec(W.shape, _w_map))
        w_specs.append(pl.BlockSpec(b.shape, _w_map))
        w_args.extend([W, b])
    grid_spec = pltpu.PrefetchScalarGridSpec(
        num_scalar_prefetch=1,
        grid=(MAXB,),
        in_specs=[
            pl.BlockSpec((BLK, D), _slot_map),
            pl.BlockSpec(memory_space=pl.ANY),
        ] + w_specs,
        out_specs=pl.BlockSpec((BLK, D), _slot_map),
    )
    return pl.pallas_call(
        _FFN_BODIES[ep_type],
        grid_spec=grid_spec,
        out_shape=jax.ShapeDtypeStruct((NSLOT, D), jnp.float32),
        input_output_aliases={2: 0},
    )(info, xs, ys, *w_args)


def _combine_body(ys_hbm, ts0_hbm, ts1_hbm, v0_hbm, v1_hbm, out_hbm,
                  idx0, idx1, buf0, buf1, vb0, vb1, sem):
    wid = lax.axis_index("s") * NC + lax.axis_index("c")
    base = wid * TPW
    pltpu.sync_copy(ts0_hbm.at[pl.ds(base, TPW)], idx0)
    pltpu.sync_copy(ts1_hbm.at[pl.ds(base, TPW)], idx1)
    pltpu.async_copy(ys_hbm.at[idx0], buf0, sem).wait()
    pltpu.async_copy(ys_hbm.at[idx1], buf1, sem).wait()
    pltpu.sync_copy(v0_hbm.at[pl.ds(base, TPW)], vb0)
    pltpu.sync_copy(v1_hbm.at[pl.ds(base, TPW)], vb1)

    def row(t, carry):
        v0 = vb0[t]
        v1 = vb1[t]
        for c in range(ROW16):
            s = c * 16
            buf0[t, s:s + 16] = (v0 * buf0[t, s:s + 16]
                                 + v1 * buf1[t, s:s + 16])
        return carry

    lax.fori_loop(0, TPW, row, 0)
    pltpu.sync_copy(buf0, out_hbm.at[pl.ds(base, TPW)])


def _run_combine(ys, ts0, ts1, v0, v1):
    mesh = plsc.VectorSubcoreMesh(core_axis_name="c", subcore_axis_name="s")
    f = pl.kernel(
        _combine_body,
        out_type=jax.ShapeDtypeStruct((T, D), jnp.float32),
        mesh=mesh,
        scratch_types=[
            pltpu.VMEM((TPW,), jnp.int32),
            pltpu.VMEM((TPW,), jnp.int32),
            pltpu.VMEM((TPW, D), jnp.float32),
            pltpu.VMEM((TPW, D), jnp.float32),
            pltpu.VMEM((TPW, 16), jnp.float32),
            pltpu.VMEM((TPW, 16), jnp.float32),
            pltpu.SemaphoreType.DMA,
        ],
    )
    return f(ys, ts0, ts1, v0, v1)


def kernel(x, params):
    Bb, Ll, Dm = x.shape
    xf = x.reshape(T, D)
    r = params['router']
    temp = params['temp'][0]
    gate = params['gate']
    w2p = r['w2'] * (gate / temp)[:, None]
    b2p = (r['b2'] / temp + params['bias']) * gate

    ts0_2d, ts1_2d, v0, v1, binfo = pl.pallas_call(
        _router_body,
        out_shape=(
            jax.ShapeDtypeStruct((T, 1), jnp.int32),
            jax.ShapeDtypeStruct((T, 1), jnp.int32),
            jax.ShapeDtypeStruct((T, 16), jnp.float32),
            jax.ShapeDtypeStruct((T, 16), jnp.float32),
            jax.ShapeDtypeStruct((E, 2), jnp.int32),
        ),
    )(xf, r['w1'], r['b1'].reshape(1, -1), r['gamma'].reshape(1, -1),
      r['beta'].reshape(1, -1), w2p, b2p.reshape(1, -1))

    ts0 = ts0_2d.reshape(T)
    ts1 = ts1_2d.reshape(T)

    xs = _run_dispatch(xf, ts0, ts1)

    ys = jnp.zeros((NSLOT, D), jnp.float32)
    for e, ep in enumerate(params['experts']):
        t = e % 3
        if t == 0:
            weights = [(ep['l1'][0], ep['l1'][1].reshape(1, -1)),
                       (ep['l2'][0], ep['l2'][1].reshape(1, -1))]
        else:
            weights = [(ep['l1'][0], ep['l1'][1].reshape(1, -1)),
                       (ep['l2'][0], ep['l2'][1].reshape(1, -1)),
                       (ep['l3'][0], ep['l3'][1].reshape(1, -1))]
        ys = _run_ffn(t, xs, ys, binfo[e], weights)

    out = _run_combine(ys, ts0, ts1, v0, v1)
    return out.reshape(Bb, Ll, Dm)
